# baseline (speedup 1.0000x reference)
import functools

import numpy as np
import jax
import jax.numpy as jnp
from jax.experimental import pallas as pl
from jax.experimental.pallas import tpu as pltpu

_L = 512
_TOPK = 128
_D_RBF = 64
_ROWS = 8


def _ln(x, g, b, eps=1e-5):
    m = jnp.mean(x, -1, keepdims=True)
    v = jnp.var(x, -1, keepdims=True)
    return (x - m) * jax.lax.rsqrt(v + eps) * g + b


def _prep_body(msa0_ref, state_ref, cas_ref, casT_ref,
               nmg_ref, nmb_ref, nsg_ref, nsb_ref,
               wn_msa_ref, wn_state_ref, bn_ref,
               ffn_ng_ref, ffn_nb_ref, ffn_w1_ref, ffn_b1_ref,
               ffn_w2_ref, ffn_b2_ref, nng_ref, nnb_ref,
               wm_src_ref, wm_dst_ref, bm_ref,
               nw1_ref, nw2_ref, d_ref):
    seq = _ln(msa0_ref[...], nmg_ref[...], nmb_ref[...])
    stn = _ln(state_ref[...], nsg_ref[...], nsb_ref[...])
    node = (jnp.dot(seq, wn_msa_ref[...], preferred_element_type=jnp.float32, precision=jax.lax.Precision.HIGHEST)
            + jnp.dot(stn, wn_state_ref[...], preferred_element_type=jnp.float32, precision=jax.lax.Precision.HIGHEST)
            + bn_ref[...])
    h = _ln(node, ffn_ng_ref[...], ffn_nb_ref[...])
    h = jax.nn.relu(jnp.dot(h, ffn_w1_ref[...], preferred_element_type=jnp.float32, precision=jax.lax.Precision.HIGHEST)
                    + ffn_b1_ref[...])
    node = node + jnp.dot(h, ffn_w2_ref[...], preferred_element_type=jnp.float32, precision=jax.lax.Precision.HIGHEST) + ffn_b2_ref[...]
    node = _ln(node, nng_ref[...], nnb_ref[...])
    nw1_ref[...] = jnp.dot(node, wm_src_ref[...], preferred_element_type=jnp.float32, precision=jax.lax.Precision.HIGHEST)
    nw2_ref[...] = jnp.dot(node, wm_dst_ref[...], preferred_element_type=jnp.float32, precision=jax.lax.Precision.HIGHEST) + bm_ref[...]

    cas = cas_ref[...]
    casT = casT_ref[...]
    d2 = jnp.zeros((_L, _L), jnp.float32)
    for c in range(3):
        diff = cas[:, c:c + 1] - casT[c:c + 1, :]
        d2 = d2 + diff * diff
    d_ref[...] = jnp.sqrt(d2 + 1e-8)


def _main_body(pair_ref, dcol_ref, nbr_ref, msa0_ref, p_ref, nw1_ref, nw2_ref,
               pg_ref, pb_ref,
               wp_ref, wr_ref, wnb_ref, be_ref,
               fe_ng_ref, fe_nb_ref, fe_w1_ref, fe_b1_ref, fe_w2_ref, fe_b2_ref,
               neg_ref, neb_ref,
               wm_edge_ref,
               wl0_ref, bl0_ref, wc_ref, bc_ref,
               s0g_ref, s0b_ref, sig_ref, sib_ref,
               ws0_ref, bs0_ref, wsi_ref, bsi_ref,
               w1_ref, b1_ref, w2_ref, b2_ref, w3_ref, b3_ref, w4_ref, b4_ref,
               wo_ref, bo_ref,
               state_out_ref, xyz_out_ref, alpha_out_ref, quat_out_ref):
    pid = pl.program_id(0)
    R = _ROWS

    pairf = pair_ref[0].reshape(R * _L, 128)
    pairn = _ln(pairf, pg_ref[...], pb_ref[...])

    df = dcol_ref[...]
    mu = (jax.lax.broadcasted_iota(jnp.int32, (1, _D_RBF), 1).astype(jnp.float32)
          * np.float32(20.0 / (_D_RBF - 1)) + 2.0)
    sigma = (22.0 - 2.0) / _D_RBF
    z = (df - mu) / sigma
    rbf = jnp.exp(-z * z)

    rr = jax.lax.broadcasted_iota(jnp.int32, (R * _L, 1), 0)
    ii = (rr >> 9) + R * pid
    jj = rr & (_L - 1)
    sep = (jj - ii).astype(jnp.float32)
    nbh = jnp.sign(sep) * jnp.log(jnp.abs(sep) + 1.0)

    e1 = (jnp.dot(pairn, wp_ref[...], preferred_element_type=jnp.float32, precision=jax.lax.Precision.HIGHEST)
          + jnp.dot(rbf, wr_ref[...], preferred_element_type=jnp.float32, precision=jax.lax.Precision.HIGHEST)
          + nbh * wnb_ref[...]
          + be_ref[...])
    he = _ln(e1, fe_ng_ref[...], fe_nb_ref[...])
    he = jax.nn.relu(jnp.dot(he, fe_w1_ref[...], preferred_element_type=jnp.float32, precision=jax.lax.Precision.HIGHEST)
                     + fe_b1_ref[...])
    e2 = e1 + jnp.dot(he, fe_w2_ref[...], preferred_element_type=jnp.float32, precision=jax.lax.Precision.HIGHEST) + fe_b2_ref[...]
    edge = _ln(e2, neg_ref[...], neb_ref[...])

    h3 = jnp.dot(edge, wm_edge_ref[...], preferred_element_type=jnp.float32, precision=jax.lax.Precision.HIGHEST)
    nw2b = nw2_ref[pl.ds(pid * R, R), :]
    h = jax.nn.relu(h3.reshape(R, _L, 64)
                    + nw1_ref[...][None, :, :]
                    + nw2b[:, None, :])

    nbrb = nbr_ref[...]
    jidx = jax.lax.broadcasted_iota(jnp.int32, (R, _TOPK, _L), 2)
    mask = jnp.sum((nbrb[:, :, None] == jidx).astype(jnp.float32), axis=1)

    hm = h * mask[:, :, None]
    hsum = jnp.sum(hm, axis=1)
    l0 = (jnp.dot(hsum, wl0_ref[...], preferred_element_type=jnp.float32, precision=jax.lax.Precision.HIGHEST)
          * (1.0 / _TOPK) + bl0_ref[...])
    state_out_ref[...] = l0

    coef = (jnp.dot(h.reshape(R * _L, 64), wc_ref[...], preferred_element_type=jnp.float32, precision=jax.lax.Precision.HIGHEST)
            + bc_ref[...]) * 0.1
    A = coef.reshape(R, _L, 8) * mask[:, :, None]

    p_all = p_ref[...]
    pblk = p_ref[pl.ds(pid * R, R), :]
    ca_i = pblk[:, 9:12]

    msum = []
    for e in range(8):
        a = e % 4
        pe = p_all[:, a * 3:a * 3 + 3]
        msum.append(jnp.dot(A[:, :, e], pe, preferred_element_type=jnp.float32, precision=jax.lax.Precision.HIGHEST))
    sA = jnp.sum(A, axis=1)
    off = []
    for k in range(2):
        acc = msum[k * 4 + 0] + msum[k * 4 + 1] + msum[k * 4 + 2] + msum[k * 4 + 3]
        acc = acc - ca_i * sA[:, k * 4 + 3:k * 4 + 4]
        off.append(acc * (1.0 / _TOPK))
    T = off[0] * (1.0 / 10.0)
    Rv = off[1] * (1.0 / 100.0)

    qn = jnp.sqrt(1.0 + jnp.sum(Rv * Rv, axis=1, keepdims=True))
    qA = 1.0 / qn
    qB = Rv[:, 0:1] / qn
    qC = Rv[:, 1:2] / qn
    qD = Rv[:, 2:3] / qn
    r = [[qA * qA + qB * qB - qC * qC - qD * qD, 2 * qB * qC - 2 * qA * qD, 2 * qB * qD + 2 * qA * qC],
         [2 * qB * qC + 2 * qA * qD, qA * qA - qB * qB + qC * qC - qD * qD, 2 * qC * qD - 2 * qA * qB],
         [2 * qB * qD - 2 * qA * qC, 2 * qC * qD + 2 * qA * qB, qA * qA - qB * qB - qC * qC + qD * qD]]
    v = pblk[:, 0:9]
    cols = []
    for a in range(3):
        for c in range(3):
            acc = (r[c][0] * v[:, a * 3 + 0:a * 3 + 1]
                   + r[c][1] * v[:, a * 3 + 1:a * 3 + 2]
                   + r[c][2] * v[:, a * 3 + 2:a * 3 + 3])
            cols.append(acc + ca_i[:, c:c + 1] + T[:, c:c + 1])
    xyz_out_ref[...] = jnp.concatenate(cols, axis=1)
    quat_out_ref[...] = jnp.concatenate([qA, qB, qC, qD], axis=1)

    s = _ln(msa0_ref[...], s0g_ref[...], s0b_ref[...])
    st = _ln(l0, sig_ref[...], sib_ref[...])
    si = (jnp.dot(s, ws0_ref[...], preferred_element_type=jnp.float32, precision=jax.lax.Precision.HIGHEST) + bs0_ref[...]
          + jnp.dot(st, wsi_ref[...], preferred_element_type=jnp.float32, precision=jax.lax.Precision.HIGHEST) + bsi_ref[...])
    t = jax.nn.relu(si)
    t = jax.nn.relu(jnp.dot(t, w1_ref[...], preferred_element_type=jnp.float32, precision=jax.lax.Precision.HIGHEST) + b1_ref[...])
    si = si + jnp.dot(t, w2_ref[...], preferred_element_type=jnp.float32, precision=jax.lax.Precision.HIGHEST) + b2_ref[...]
    t = jax.nn.relu(si)
    t = jax.nn.relu(jnp.dot(t, w3_ref[...], preferred_element_type=jnp.float32, precision=jax.lax.Precision.HIGHEST) + b3_ref[...])
    si = si + jnp.dot(t, w4_ref[...], preferred_element_type=jnp.float32, precision=jax.lax.Precision.HIGHEST) + b4_ref[...]
    alpha_out_ref[...] = (jnp.dot(jax.nn.relu(si), wo_ref[...],
                                  preferred_element_type=jnp.float32, precision=jax.lax.Precision.HIGHEST) + bo_ref[...])


def _full(shape):
    return pl.BlockSpec(shape, lambda i: tuple(0 for _ in shape))


def kernel(msa, pair, xyz, state, idx, rotation_mask, bond_feats, dist_matrix,
           atom_frames, is_motif, params):
    p = params
    msa0 = msa[0, 0]
    state0 = state[0]
    xyzf = xyz[0]
    cas = xyzf[:, 1, :]
    casT = jnp.transpose(cas)
    l1 = (xyzf - cas[:, None, :]).reshape(_L, 9)
    P = jnp.concatenate([l1, cas], axis=1)

    wn = p["embed_node"]["w"]
    wm = p["se3_msg"]["w"]
    we = p["embed_edge"]["w"]

    prep_in = (
        msa0, state0, cas, casT,
        p["norm_msa"]["g"], p["norm_msa"]["b"],
        p["norm_state"]["g"], p["norm_state"]["b"],
        wn[:256], wn[256:], p["embed_node"]["b"],
        p["ff_node"]["ng"], p["ff_node"]["nb"],
        p["ff_node"]["l1"]["w"], p["ff_node"]["l1"]["b"],
        p["ff_node"]["l2"]["w"], p["ff_node"]["l2"]["b"],
        p["norm_node"]["g"], p["norm_node"]["b"],
        wm[0:32], wm[32:64], p["se3_msg"]["b"],
    )
    nw1, nw2, D = pl.pallas_call(
        _prep_body,
        out_shape=(
            jax.ShapeDtypeStruct((_L, 64), jnp.float32),
            jax.ShapeDtypeStruct((_L, 64), jnp.float32),
            jax.ShapeDtypeStruct((_L, _L), jnp.float32),
        ),
    )(*prep_in)

    cas_b = xyz[:, :, 1]
    d2_sel = jnp.sum(jnp.square(cas_b[:, :, None, :] - cas_b[:, None, :, :]), -1)
    dg_sel = jnp.sqrt(d2_sel + 1e-8)[0] + jnp.eye(_L) * 1e6
    _, nbr = jax.lax.top_k(-dg_sel, _TOPK)
    nbr = nbr.astype(jnp.int32)

    R = _ROWS
    grid = (_L // R,)
    main_in_specs = [
        pl.BlockSpec((1, R, _L, 128), lambda i: (0, i, 0, 0)),
        pl.BlockSpec((R * _L, 1), lambda i: (i, 0)),
        pl.BlockSpec((R, _TOPK), lambda i: (i, 0)),
        pl.BlockSpec((R, 256), lambda i: (i, 0)),
        _full((_L, 12)),
        _full((_L, 64)), _full((_L, 64)),
        _full((128,)), _full((128,)),
        _full((128, 32)), _full((64, 32)), _full((1, 32)), _full((32,)),
        _full((32,)), _full((32,)), _full((32, 64)), _full((64,)),
        _full((64, 32)), _full((32,)),
        _full((32,)), _full((32,)),
        _full((32, 64)),
        _full((64, 32)), _full((32,)), _full((64, 8)), _full((8,)),
        _full((256,)), _full((256,)), _full((32,)), _full((32,)),
        _full((256, 128)), _full((128,)), _full((32, 128)), _full((128,)),
        _full((128, 128)), _full((128,)), _full((128, 128)), _full((128,)),
        _full((128, 128)), _full((128,)), _full((128, 128)), _full((128,)),
        _full((128, 20)), _full((20,)),
    ]
    main_out_specs = (
        pl.BlockSpec((R, 32), lambda i: (i, 0)),
        pl.BlockSpec((R, 9), lambda i: (i, 0)),
        pl.BlockSpec((R, 20), lambda i: (i, 0)),
        pl.BlockSpec((R, 4), lambda i: (i, 0)),
    )
    main_in = (
        pair, D.reshape(_L * _L, 1), nbr, msa0, P, nw1, nw2,
        p["norm_pair"]["g"], p["norm_pair"]["b"],
        we[:128], we[128:192], we[192:193], p["embed_edge"]["b"],
        p["ff_edge"]["ng"], p["ff_edge"]["nb"],
        p["ff_edge"]["l1"]["w"], p["ff_edge"]["l1"]["b"],
        p["ff_edge"]["l2"]["w"], p["ff_edge"]["l2"]["b"],
        p["norm_edge"]["g"], p["norm_edge"]["b"],
        wm[64:96],
        p["se3_l0"]["w"], p["se3_l0"]["b"], p["se3_coef"]["w"], p["se3_coef"]["b"],
        p["sc_norm_s0"]["g"], p["sc_norm_s0"]["b"],
        p["sc_norm_si"]["g"], p["sc_norm_si"]["b"],
        p["sc_s0"]["w"], p["sc_s0"]["b"], p["sc_si"]["w"], p["sc_si"]["b"],
        p["sc_1"]["w"], p["sc_1"]["b"], p["sc_2"]["w"], p["sc_2"]["b"],
        p["sc_3"]["w"], p["sc_3"]["b"], p["sc_4"]["w"], p["sc_4"]["b"],
        p["sc_out"]["w"], p["sc_out"]["b"],
    )
    state_new, xyz9, alpha, quat = pl.pallas_call(
        _main_body,
        grid=grid,
        in_specs=main_in_specs,
        out_specs=main_out_specs,
        out_shape=(
            jax.ShapeDtypeStruct((_L, 32), jnp.float32),
            jax.ShapeDtypeStruct((_L, 9), jnp.float32),
            jax.ShapeDtypeStruct((_L, 20), jnp.float32),
            jax.ShapeDtypeStruct((_L, 4), jnp.float32),
        ),
        compiler_params=pltpu.CompilerParams(
            dimension_semantics=("arbitrary",),
        ),
    )(*main_in)

    xyz_new = xyz9.reshape(1, _L, 3, 3)
    state_out = state_new.reshape(1, _L, 32)
    alpha_out = alpha.reshape(1, _L, 10, 2)
    quat_out = quat.reshape(1, _L, 4)
    return xyz_new, state_out, alpha_out, quat_out


# speedup vs baseline: 2.6139x; 2.6139x over previous
import functools

import numpy as np
import jax
import jax.numpy as jnp
from jax.experimental import pallas as pl
from jax.experimental.pallas import tpu as pltpu

_L = 512
_TOPK = 128
_D_RBF = 64
_ROWS = 8


def _ln(x, g, b, eps=1e-5):
    m = jnp.mean(x, -1, keepdims=True)
    v = jnp.var(x, -1, keepdims=True)
    return (x - m) * jax.lax.rsqrt(v + eps) * g + b


def _prep_body(msa0_ref, state_ref, cas_ref, casT_ref,
               nmg_ref, nmb_ref, nsg_ref, nsb_ref,
               wn_msa_ref, wn_state_ref, bn_ref,
               ffn_ng_ref, ffn_nb_ref, ffn_w1_ref, ffn_b1_ref,
               ffn_w2_ref, ffn_b2_ref, nng_ref, nnb_ref,
               wm_src_ref, wm_dst_ref, bm_ref,
               nw1_ref, nw2_ref, d_ref):
    seq = _ln(msa0_ref[...], nmg_ref[...], nmb_ref[...])
    stn = _ln(state_ref[...], nsg_ref[...], nsb_ref[...])
    node = (jnp.dot(seq, wn_msa_ref[...], preferred_element_type=jnp.float32)
            + jnp.dot(stn, wn_state_ref[...], preferred_element_type=jnp.float32)
            + bn_ref[...])
    h = _ln(node, ffn_ng_ref[...], ffn_nb_ref[...])
    h = jax.nn.relu(jnp.dot(h, ffn_w1_ref[...], preferred_element_type=jnp.float32)
                    + ffn_b1_ref[...])
    node = node + jnp.dot(h, ffn_w2_ref[...], preferred_element_type=jnp.float32) + ffn_b2_ref[...]
    node = _ln(node, nng_ref[...], nnb_ref[...])
    nw1_ref[...] = jnp.dot(node, wm_src_ref[...], preferred_element_type=jnp.float32)
    nw2_ref[...] = jnp.dot(node, wm_dst_ref[...], preferred_element_type=jnp.float32) + bm_ref[...]

    cas = cas_ref[...]
    casT = casT_ref[...]
    d2 = jnp.zeros((_L, _L), jnp.float32)
    for c in range(3):
        diff = cas[:, c:c + 1] - casT[c:c + 1, :]
        d2 = d2 + diff * diff
    d_ref[...] = jnp.sqrt(d2 + 1e-8)


def _main_body(pair_ref, dcol_ref, nbr_ref, msa0_ref, p_ref, nw1_ref, nw2_ref,
               pg_ref, pb_ref,
               wp_ref, wr_ref, wnb_ref, be_ref,
               fe_ng_ref, fe_nb_ref, fe_w1_ref, fe_b1_ref, fe_w2_ref, fe_b2_ref,
               neg_ref, neb_ref,
               wm_edge_ref,
               wl0_ref, bl0_ref, wc_ref, bc_ref,
               s0g_ref, s0b_ref, sig_ref, sib_ref,
               ws0_ref, bs0_ref, wsi_ref, bsi_ref,
               w1_ref, b1_ref, w2_ref, b2_ref, w3_ref, b3_ref, w4_ref, b4_ref,
               wo_ref, bo_ref,
               state_out_ref, xyz_out_ref, alpha_out_ref, quat_out_ref):
    pid = pl.program_id(0)
    R = _ROWS

    pairf = pair_ref[0].reshape(R * _L, 128)
    pairn = _ln(pairf, pg_ref[...], pb_ref[...])

    df = dcol_ref[...]
    mu = (jax.lax.broadcasted_iota(jnp.int32, (1, _D_RBF), 1).astype(jnp.float32)
          * np.float32(20.0 / (_D_RBF - 1)) + 2.0)
    sigma = (22.0 - 2.0) / _D_RBF
    z = (df - mu) / sigma
    rbf = jnp.exp(-z * z)

    rr = jax.lax.broadcasted_iota(jnp.int32, (R * _L, 1), 0)
    ii = (rr >> 9) + R * pid
    jj = rr & (_L - 1)
    sep = (jj - ii).astype(jnp.float32)
    nbh = jnp.sign(sep) * jnp.log(jnp.abs(sep) + 1.0)

    e1 = (jnp.dot(pairn, wp_ref[...], preferred_element_type=jnp.float32)
          + jnp.dot(rbf, wr_ref[...], preferred_element_type=jnp.float32)
          + nbh * wnb_ref[...]
          + be_ref[...])
    he = _ln(e1, fe_ng_ref[...], fe_nb_ref[...])
    he = jax.nn.relu(jnp.dot(he, fe_w1_ref[...], preferred_element_type=jnp.float32)
                     + fe_b1_ref[...])
    e2 = e1 + jnp.dot(he, fe_w2_ref[...], preferred_element_type=jnp.float32) + fe_b2_ref[...]
    edge = _ln(e2, neg_ref[...], neb_ref[...])

    h3 = jnp.dot(edge, wm_edge_ref[...], preferred_element_type=jnp.float32)
    nw2b = nw2_ref[pl.ds(pid * R, R), :]
    h = jax.nn.relu(h3.reshape(R, _L, 64)
                    + nw1_ref[...][None, :, :]
                    + nw2b[:, None, :])

    nbrb = nbr_ref[...]
    jidx = jax.lax.broadcasted_iota(jnp.int32, (R, _TOPK, _L), 2)
    mask = jnp.sum((nbrb[:, :, None] == jidx).astype(jnp.float32), axis=1)

    hm = h * mask[:, :, None]
    hsum = jnp.sum(hm, axis=1)
    l0 = (jnp.dot(hsum, wl0_ref[...], preferred_element_type=jnp.float32)
          * (1.0 / _TOPK) + bl0_ref[...])
    state_out_ref[...] = l0

    coef = (jnp.dot(h.reshape(R * _L, 64), wc_ref[...], preferred_element_type=jnp.float32)
            + bc_ref[...]) * 0.1
    A = coef.reshape(R, _L, 8) * mask[:, :, None]

    p_all = p_ref[...]
    pblk = p_ref[pl.ds(pid * R, R), :]
    ca_i = pblk[:, 9:12]

    msum = []
    for e in range(8):
        a = e % 4
        pe = p_all[:, a * 3:a * 3 + 3]
        msum.append(jnp.dot(A[:, :, e], pe, preferred_element_type=jnp.float32))
    sA = jnp.sum(A, axis=1)
    off = []
    for k in range(2):
        acc = msum[k * 4 + 0] + msum[k * 4 + 1] + msum[k * 4 + 2] + msum[k * 4 + 3]
        acc = acc - ca_i * sA[:, k * 4 + 3:k * 4 + 4]
        off.append(acc * (1.0 / _TOPK))
    T = off[0] * (1.0 / 10.0)
    Rv = off[1] * (1.0 / 100.0)

    qn = jnp.sqrt(1.0 + jnp.sum(Rv * Rv, axis=1, keepdims=True))
    qA = 1.0 / qn
    qB = Rv[:, 0:1] / qn
    qC = Rv[:, 1:2] / qn
    qD = Rv[:, 2:3] / qn
    r = [[qA * qA + qB * qB - qC * qC - qD * qD, 2 * qB * qC - 2 * qA * qD, 2 * qB * qD + 2 * qA * qC],
         [2 * qB * qC + 2 * qA * qD, qA * qA - qB * qB + qC * qC - qD * qD, 2 * qC * qD - 2 * qA * qB],
         [2 * qB * qD - 2 * qA * qC, 2 * qC * qD + 2 * qA * qB, qA * qA - qB * qB - qC * qC + qD * qD]]
    v = pblk[:, 0:9]
    cols = []
    for a in range(3):
        for c in range(3):
            acc = (r[c][0] * v[:, a * 3 + 0:a * 3 + 1]
                   + r[c][1] * v[:, a * 3 + 1:a * 3 + 2]
                   + r[c][2] * v[:, a * 3 + 2:a * 3 + 3])
            cols.append(acc + ca_i[:, c:c + 1] + T[:, c:c + 1])
    xyz_out_ref[...] = jnp.concatenate(cols, axis=1)
    quat_out_ref[...] = jnp.concatenate([qA, qB, qC, qD], axis=1)

    s = _ln(msa0_ref[...], s0g_ref[...], s0b_ref[...])
    st = _ln(l0, sig_ref[...], sib_ref[...])
    si = (jnp.dot(s, ws0_ref[...], preferred_element_type=jnp.float32) + bs0_ref[...]
          + jnp.dot(st, wsi_ref[...], preferred_element_type=jnp.float32) + bsi_ref[...])
    t = jax.nn.relu(si)
    t = jax.nn.relu(jnp.dot(t, w1_ref[...], preferred_element_type=jnp.float32) + b1_ref[...])
    si = si + jnp.dot(t, w2_ref[...], preferred_element_type=jnp.float32) + b2_ref[...]
    t = jax.nn.relu(si)
    t = jax.nn.relu(jnp.dot(t, w3_ref[...], preferred_element_type=jnp.float32) + b3_ref[...])
    si = si + jnp.dot(t, w4_ref[...], preferred_element_type=jnp.float32) + b4_ref[...]
    alpha_out_ref[...] = (jnp.dot(jax.nn.relu(si), wo_ref[...],
                                  preferred_element_type=jnp.float32) + bo_ref[...])


def _full(shape):
    return pl.BlockSpec(shape, lambda i: tuple(0 for _ in shape))


def kernel(msa, pair, xyz, state, idx, rotation_mask, bond_feats, dist_matrix,
           atom_frames, is_motif, params):
    p = params
    msa0 = msa[0, 0]
    state0 = state[0]
    xyzf = xyz[0]
    cas = xyzf[:, 1, :]
    casT = jnp.transpose(cas)
    l1 = (xyzf - cas[:, None, :]).reshape(_L, 9)
    P = jnp.concatenate([l1, cas], axis=1)

    wn = p["embed_node"]["w"]
    wm = p["se3_msg"]["w"]
    we = p["embed_edge"]["w"]

    prep_in = (
        msa0, state0, cas, casT,
        p["norm_msa"]["g"], p["norm_msa"]["b"],
        p["norm_state"]["g"], p["norm_state"]["b"],
        wn[:256], wn[256:], p["embed_node"]["b"],
        p["ff_node"]["ng"], p["ff_node"]["nb"],
        p["ff_node"]["l1"]["w"], p["ff_node"]["l1"]["b"],
        p["ff_node"]["l2"]["w"], p["ff_node"]["l2"]["b"],
        p["norm_node"]["g"], p["norm_node"]["b"],
        wm[0:32], wm[32:64], p["se3_msg"]["b"],
    )
    nw1, nw2, D = pl.pallas_call(
        _prep_body,
        out_shape=(
            jax.ShapeDtypeStruct((_L, 64), jnp.float32),
            jax.ShapeDtypeStruct((_L, 64), jnp.float32),
            jax.ShapeDtypeStruct((_L, _L), jnp.float32),
        ),
    )(*prep_in)

    cas_b = xyz[:, :, 1]
    d2_sel = jnp.sum(jnp.square(cas_b[:, :, None, :] - cas_b[:, None, :, :]), -1)
    dg_sel = jnp.sqrt(d2_sel + 1e-8)[0] + jnp.eye(_L) * 1e6
    _, nbr = jax.lax.top_k(-dg_sel, _TOPK)
    nbr = nbr.astype(jnp.int32)

    R = _ROWS
    grid = (_L // R,)
    main_in_specs = [
        pl.BlockSpec((1, R, _L, 128), lambda i: (0, i, 0, 0)),
        pl.BlockSpec((R * _L, 1), lambda i: (i, 0)),
        pl.BlockSpec((R, _TOPK), lambda i: (i, 0)),
        pl.BlockSpec((R, 256), lambda i: (i, 0)),
        _full((_L, 12)),
        _full((_L, 64)), _full((_L, 64)),
        _full((128,)), _full((128,)),
        _full((128, 32)), _full((64, 32)), _full((1, 32)), _full((32,)),
        _full((32,)), _full((32,)), _full((32, 64)), _full((64,)),
        _full((64, 32)), _full((32,)),
        _full((32,)), _full((32,)),
        _full((32, 64)),
        _full((64, 32)), _full((32,)), _full((64, 8)), _full((8,)),
        _full((256,)), _full((256,)), _full((32,)), _full((32,)),
        _full((256, 128)), _full((128,)), _full((32, 128)), _full((128,)),
        _full((128, 128)), _full((128,)), _full((128, 128)), _full((128,)),
        _full((128, 128)), _full((128,)), _full((128, 128)), _full((128,)),
        _full((128, 20)), _full((20,)),
    ]
    main_out_specs = (
        pl.BlockSpec((R, 32), lambda i: (i, 0)),
        pl.BlockSpec((R, 9), lambda i: (i, 0)),
        pl.BlockSpec((R, 20), lambda i: (i, 0)),
        pl.BlockSpec((R, 4), lambda i: (i, 0)),
    )
    main_in = (
        pair, D.reshape(_L * _L, 1), nbr, msa0, P, nw1, nw2,
        p["norm_pair"]["g"], p["norm_pair"]["b"],
        we[:128], we[128:192], we[192:193], p["embed_edge"]["b"],
        p["ff_edge"]["ng"], p["ff_edge"]["nb"],
        p["ff_edge"]["l1"]["w"], p["ff_edge"]["l1"]["b"],
        p["ff_edge"]["l2"]["w"], p["ff_edge"]["l2"]["b"],
        p["norm_edge"]["g"], p["norm_edge"]["b"],
        wm[64:96],
        p["se3_l0"]["w"], p["se3_l0"]["b"], p["se3_coef"]["w"], p["se3_coef"]["b"],
        p["sc_norm_s0"]["g"], p["sc_norm_s0"]["b"],
        p["sc_norm_si"]["g"], p["sc_norm_si"]["b"],
        p["sc_s0"]["w"], p["sc_s0"]["b"], p["sc_si"]["w"], p["sc_si"]["b"],
        p["sc_1"]["w"], p["sc_1"]["b"], p["sc_2"]["w"], p["sc_2"]["b"],
        p["sc_3"]["w"], p["sc_3"]["b"], p["sc_4"]["w"], p["sc_4"]["b"],
        p["sc_out"]["w"], p["sc_out"]["b"],
    )
    state_new, xyz9, alpha, quat = pl.pallas_call(
        _main_body,
        grid=grid,
        in_specs=main_in_specs,
        out_specs=main_out_specs,
        out_shape=(
            jax.ShapeDtypeStruct((_L, 32), jnp.float32),
            jax.ShapeDtypeStruct((_L, 9), jnp.float32),
            jax.ShapeDtypeStruct((_L, 20), jnp.float32),
            jax.ShapeDtypeStruct((_L, 4), jnp.float32),
        ),
        compiler_params=pltpu.CompilerParams(
            dimension_semantics=("arbitrary",),
        ),
    )(*main_in)

    xyz_new = xyz9.reshape(1, _L, 3, 3)
    state_out = state_new.reshape(1, _L, 32)
    alpha_out = alpha.reshape(1, _L, 10, 2)
    quat_out = quat.reshape(1, _L, 4)
    return xyz_new, state_out, alpha_out, quat_out


# speedup vs baseline: 3.2299x; 1.2356x over previous
import functools

import numpy as np
import jax
import jax.numpy as jnp
from jax.experimental import pallas as pl
from jax.experimental.pallas import tpu as pltpu

_L = 512
_TOPK = 128
_D_RBF = 64
_ROWS = 16


def _ln(x, g, b, eps=1e-5):
    m = jnp.mean(x, -1, keepdims=True)
    v = jnp.var(x, -1, keepdims=True)
    return (x - m) * jax.lax.rsqrt(v + eps) * g + b


def _prep_body(msa0_ref, state_ref,
               nmg_ref, nmb_ref, nsg_ref, nsb_ref,
               wn_msa_ref, wn_state_ref, bn_ref,
               ffn_ng_ref, ffn_nb_ref, ffn_w1_ref, ffn_b1_ref,
               ffn_w2_ref, ffn_b2_ref, nng_ref, nnb_ref,
               wm_src_ref, wm_dst_ref, bm_ref,
               nw1_ref, nw2_ref):
    seq = _ln(msa0_ref[...], nmg_ref[...], nmb_ref[...])
    stn = _ln(state_ref[...], nsg_ref[...], nsb_ref[...])
    node = (jnp.dot(seq, wn_msa_ref[...], preferred_element_type=jnp.float32)
            + jnp.dot(stn, wn_state_ref[...], preferred_element_type=jnp.float32)
            + bn_ref[...])
    h = _ln(node, ffn_ng_ref[...], ffn_nb_ref[...])
    h = jax.nn.relu(jnp.dot(h, ffn_w1_ref[...], preferred_element_type=jnp.float32)
                    + ffn_b1_ref[...])
    node = node + jnp.dot(h, ffn_w2_ref[...], preferred_element_type=jnp.float32) + ffn_b2_ref[...]
    node = _ln(node, nng_ref[...], nnb_ref[...])
    nw1_ref[...] = jnp.dot(node, wm_src_ref[...], preferred_element_type=jnp.float32)
    nw2_ref[...] = jnp.dot(node, wm_dst_ref[...], preferred_element_type=jnp.float32) + bm_ref[...]


def _main_body(pair_ref, nbr_ref, msa0_ref, p_ref, tbl_ref, nw2_ref,
               pg_ref, pb_ref,
               wp_ref, wr_ref, wnb_ref, be_ref,
               fe_ng_ref, fe_nb_ref, fe_w1_ref, fe_b1_ref, fe_w2_ref, fe_b2_ref,
               neg_ref, neb_ref,
               wm_edge_ref,
               wl0_ref, bl0_ref, wc_ref, bc_ref,
               s0g_ref, s0b_ref, sig_ref, sib_ref,
               ws0_ref, bs0_ref, wsi_ref, bsi_ref,
               w1_ref, b1_ref, w2_ref, b2_ref, w3_ref, b3_ref, w4_ref, b4_ref,
               wo_ref, bo_ref,
               state_out_ref, xyz_out_ref, alpha_out_ref, quat_out_ref):
    pid = pl.program_id(0)
    R = _ROWS
    EB = R * _TOPK

    nbrb = nbr_ref[...]
    jidx = jax.lax.broadcasted_iota(jnp.int32, (R, _TOPK, _L), 2)
    oh = (nbrb[:, :, None] == jidx).astype(jnp.float32)

    pairb = pair_ref[0]
    tbl = tbl_ref[...]
    gp_parts = []
    gt_parts = []
    for i in range(R):
        ohi = oh[i]
        gp_parts.append(jnp.dot(ohi, pairb[i], preferred_element_type=jnp.float32))
        gt_parts.append(jnp.dot(ohi, tbl, preferred_element_type=jnp.float32))
    gp = jnp.concatenate(gp_parts, axis=0)
    gt = jnp.concatenate(gt_parts, axis=0)
    gl1a = gt[:, 0:3]
    gl1b = gt[:, 3:6]
    gl1c = gt[:, 6:9]
    gca = gt[:, 9:12]
    gsrc = gt[:, 12:13]
    gnw1 = gt[:, 128:192]

    pblk = p_ref[pl.ds(pid * R, R), :]
    ca_i = pblk[:, 9:12]
    ca_e = jnp.broadcast_to(ca_i[:, None, :], (R, _TOPK, 3)).reshape(EB, 3)

    rel = gca - ca_e
    d2 = (rel[:, 0:1] * rel[:, 0:1] + rel[:, 1:2] * rel[:, 1:2]
          + rel[:, 2:3] * rel[:, 2:3])
    d = jnp.sqrt(d2 + 1e-8)

    mu = (jax.lax.broadcasted_iota(jnp.int32, (1, _D_RBF), 1).astype(jnp.float32)
          * np.float32(20.0 / (_D_RBF - 1)) + 2.0)
    sigma = (22.0 - 2.0) / _D_RBF
    z = (d - mu) / sigma
    rbf = jnp.exp(-z * z)

    re = jax.lax.broadcasted_iota(jnp.int32, (EB, 1), 0)
    ii = ((re >> 7) + R * pid).astype(jnp.float32)
    sep = gsrc - ii
    nbh = jnp.sign(sep) * jnp.log(jnp.abs(sep) + 1.0)

    pairn = _ln(gp, pg_ref[...], pb_ref[...])
    e1 = (jnp.dot(pairn, wp_ref[...], preferred_element_type=jnp.float32)
          + jnp.dot(rbf, wr_ref[...], preferred_element_type=jnp.float32)
          + nbh * wnb_ref[...]
          + be_ref[...])
    he = _ln(e1, fe_ng_ref[...], fe_nb_ref[...])
    he = jax.nn.relu(jnp.dot(he, fe_w1_ref[...], preferred_element_type=jnp.float32)
                     + fe_b1_ref[...])
    e2 = e1 + jnp.dot(he, fe_w2_ref[...], preferred_element_type=jnp.float32) + fe_b2_ref[...]
    edge = _ln(e2, neg_ref[...], neb_ref[...])

    nw2b = nw2_ref[pl.ds(pid * R, R), :]
    nw2e = jnp.broadcast_to(nw2b[:, None, :], (R, _TOPK, 64)).reshape(EB, 64)
    h = jax.nn.relu(jnp.dot(edge, wm_edge_ref[...], preferred_element_type=jnp.float32)
                    + gnw1 + nw2e)

    coef = (jnp.dot(h, wc_ref[...], preferred_element_type=jnp.float32)
            + bc_ref[...]) * 0.1

    lm_cols = []
    for k in range(2):
        for c in range(3):
            acc = (coef[:, 4 * k + 0:4 * k + 1] * gl1a[:, c:c + 1]
                   + coef[:, 4 * k + 1:4 * k + 2] * gl1b[:, c:c + 1]
                   + coef[:, 4 * k + 2:4 * k + 3] * gl1c[:, c:c + 1]
                   + coef[:, 4 * k + 3:4 * k + 4] * rel[:, c:c + 1])
            lm_cols.append(acc)
    lm = jnp.concatenate(lm_cols, axis=1)

    ri = jax.lax.broadcasted_iota(jnp.int32, (R, EB), 0)
    ci = jax.lax.broadcasted_iota(jnp.int32, (R, EB), 1)
    sel = ((ci >> 7) == ri).astype(jnp.float32)
    hsum = jnp.dot(sel, h, preferred_element_type=jnp.float32)
    lsum = jnp.dot(sel, lm, preferred_element_type=jnp.float32)

    l0 = (jnp.dot(hsum, wl0_ref[...], preferred_element_type=jnp.float32)
          * (1.0 / _TOPK) + bl0_ref[...])
    state_out_ref[...] = l0

    T = lsum[:, 0:3] * (1.0 / (_TOPK * 10.0))
    Rv = lsum[:, 3:6] * (1.0 / (_TOPK * 100.0))

    qn = jnp.sqrt(1.0 + jnp.sum(Rv * Rv, axis=1, keepdims=True))
    qA = 1.0 / qn
    qB = Rv[:, 0:1] / qn
    qC = Rv[:, 1:2] / qn
    qD = Rv[:, 2:3] / qn
    r = [[qA * qA + qB * qB - qC * qC - qD * qD, 2 * qB * qC - 2 * qA * qD, 2 * qB * qD + 2 * qA * qC],
         [2 * qB * qC + 2 * qA * qD, qA * qA - qB * qB + qC * qC - qD * qD, 2 * qC * qD - 2 * qA * qB],
         [2 * qB * qD - 2 * qA * qC, 2 * qC * qD + 2 * qA * qB, qA * qA - qB * qB - qC * qC + qD * qD]]
    v = pblk[:, 0:9]
    cols = []
    for a in range(3):
        for c in range(3):
            acc = (r[c][0] * v[:, a * 3 + 0:a * 3 + 1]
                   + r[c][1] * v[:, a * 3 + 1:a * 3 + 2]
                   + r[c][2] * v[:, a * 3 + 2:a * 3 + 3])
            cols.append(acc + ca_i[:, c:c + 1] + T[:, c:c + 1])
    xyz_out_ref[...] = jnp.concatenate(cols, axis=1)
    quat_out_ref[...] = jnp.concatenate([qA, qB, qC, qD], axis=1)

    s = _ln(msa0_ref[...], s0g_ref[...], s0b_ref[...])
    st = _ln(l0, sig_ref[...], sib_ref[...])
    si = (jnp.dot(s, ws0_ref[...], preferred_element_type=jnp.float32) + bs0_ref[...]
          + jnp.dot(st, wsi_ref[...], preferred_element_type=jnp.float32) + bsi_ref[...])
    t = jax.nn.relu(si)
    t = jax.nn.relu(jnp.dot(t, w1_ref[...], preferred_element_type=jnp.float32) + b1_ref[...])
    si = si + jnp.dot(t, w2_ref[...], preferred_element_type=jnp.float32) + b2_ref[...]
    t = jax.nn.relu(si)
    t = jax.nn.relu(jnp.dot(t, w3_ref[...], preferred_element_type=jnp.float32) + b3_ref[...])
    si = si + jnp.dot(t, w4_ref[...], preferred_element_type=jnp.float32) + b4_ref[...]
    alpha_out_ref[...] = (jnp.dot(jax.nn.relu(si), wo_ref[...],
                                  preferred_element_type=jnp.float32) + bo_ref[...])


def _full(shape):
    return pl.BlockSpec(shape, lambda i: tuple(0 for _ in shape))


def kernel(msa, pair, xyz, state, idx, rotation_mask, bond_feats, dist_matrix,
           atom_frames, is_motif, params):
    p = params
    msa0 = msa[0, 0]
    state0 = state[0]
    xyzf = xyz[0]
    cas = xyzf[:, 1, :]
    l1 = (xyzf - cas[:, None, :]).reshape(_L, 9)
    P = jnp.concatenate([l1, cas], axis=1)

    wn = p["embed_node"]["w"]
    wm = p["se3_msg"]["w"]
    we = p["embed_edge"]["w"]

    prep_in = (
        msa0, state0,
        p["norm_msa"]["g"], p["norm_msa"]["b"],
        p["norm_state"]["g"], p["norm_state"]["b"],
        wn[:256], wn[256:], p["embed_node"]["b"],
        p["ff_node"]["ng"], p["ff_node"]["nb"],
        p["ff_node"]["l1"]["w"], p["ff_node"]["l1"]["b"],
        p["ff_node"]["l2"]["w"], p["ff_node"]["l2"]["b"],
        p["norm_node"]["g"], p["norm_node"]["b"],
        wm[0:32], wm[32:64], p["se3_msg"]["b"],
    )
    nw1, nw2 = pl.pallas_call(
        _prep_body,
        out_shape=(
            jax.ShapeDtypeStruct((_L, 64), jnp.float32),
            jax.ShapeDtypeStruct((_L, 64), jnp.float32),
        ),
    )(*prep_in)

    cas_b = xyz[:, :, 1]
    d2_sel = jnp.sum(jnp.square(cas_b[:, :, None, :] - cas_b[:, None, :, :]), -1)
    dg_sel = jnp.sqrt(d2_sel + 1e-8)[0] + jnp.eye(_L) * 1e6
    _, nbr = jax.lax.top_k(-dg_sel, _TOPK)
    nbr = nbr.astype(jnp.int32)

    R = _ROWS
    grid = (_L // R,)
    main_in_specs = [
        pl.BlockSpec((1, R, _L, 128), lambda i: (0, i, 0, 0)),
        pl.BlockSpec((R, _TOPK), lambda i: (i, 0)),
        pl.BlockSpec((R, 256), lambda i: (i, 0)),
        _full((_L, 12)),
        _full((_L, 256)),
        _full((_L, 64)),
        _full((128,)), _full((128,)),
        _full((128, 32)), _full((64, 32)), _full((1, 32)), _full((32,)),
        _full((32,)), _full((32,)), _full((32, 64)), _full((64,)),
        _full((64, 32)), _full((32,)),
        _full((32,)), _full((32,)),
        _full((32, 64)),
        _full((64, 32)), _full((32,)), _full((64, 8)), _full((8,)),
        _full((256,)), _full((256,)), _full((32,)), _full((32,)),
        _full((256, 128)), _full((128,)), _full((32, 128)), _full((128,)),
        _full((128, 128)), _full((128,)), _full((128, 128)), _full((128,)),
        _full((128, 128)), _full((128,)), _full((128, 128)), _full((128,)),
        _full((128, 20)), _full((20,)),
    ]
    main_out_specs = (
        pl.BlockSpec((R, 32), lambda i: (i, 0)),
        pl.BlockSpec((R, 9), lambda i: (i, 0)),
        pl.BlockSpec((R, 20), lambda i: (i, 0)),
        pl.BlockSpec((R, 4), lambda i: (i, 0)),
    )
    jcolf = jnp.arange(_L, dtype=jnp.float32)[:, None]
    small = jnp.concatenate([l1, cas, jcolf], axis=1)
    small = jnp.pad(small, ((0, 0), (0, 115)))
    tbl = jnp.concatenate([small, jnp.pad(nw1, ((0, 0), (0, 64)))],
                          axis=1)

    main_in = (
        pair, nbr, msa0, P, tbl, nw2,
        p["norm_pair"]["g"], p["norm_pair"]["b"],
        we[:128], we[128:192], we[192:193], p["embed_edge"]["b"],
        p["ff_edge"]["ng"], p["ff_edge"]["nb"],
        p["ff_edge"]["l1"]["w"], p["ff_edge"]["l1"]["b"],
        p["ff_edge"]["l2"]["w"], p["ff_edge"]["l2"]["b"],
        p["norm_edge"]["g"], p["norm_edge"]["b"],
        wm[64:96],
        p["se3_l0"]["w"], p["se3_l0"]["b"], p["se3_coef"]["w"], p["se3_coef"]["b"],
        p["sc_norm_s0"]["g"], p["sc_norm_s0"]["b"],
        p["sc_norm_si"]["g"], p["sc_norm_si"]["b"],
        p["sc_s0"]["w"], p["sc_s0"]["b"], p["sc_si"]["w"], p["sc_si"]["b"],
        p["sc_1"]["w"], p["sc_1"]["b"], p["sc_2"]["w"], p["sc_2"]["b"],
        p["sc_3"]["w"], p["sc_3"]["b"], p["sc_4"]["w"], p["sc_4"]["b"],
        p["sc_out"]["w"], p["sc_out"]["b"],
    )
    state_new, xyz9, alpha, quat = pl.pallas_call(
        _main_body,
        grid=grid,
        in_specs=main_in_specs,
        out_specs=main_out_specs,
        out_shape=(
            jax.ShapeDtypeStruct((_L, 32), jnp.float32),
            jax.ShapeDtypeStruct((_L, 9), jnp.float32),
            jax.ShapeDtypeStruct((_L, 20), jnp.float32),
            jax.ShapeDtypeStruct((_L, 4), jnp.float32),
        ),
        compiler_params=pltpu.CompilerParams(
            dimension_semantics=("arbitrary",),
        ),
    )(*main_in)

    xyz_new = xyz9.reshape(1, _L, 3, 3)
    state_out = state_new.reshape(1, _L, 32)
    alpha_out = alpha.reshape(1, _L, 10, 2)
    quat_out = quat.reshape(1, _L, 4)
    return xyz_new, state_out, alpha_out, quat_out


# speedup vs baseline: 7.5121x; 2.3258x over previous
import functools

import numpy as np
import jax
import jax.numpy as jnp
from jax.experimental import pallas as pl
from jax.experimental.pallas import tpu as pltpu

_L = 512
_TOPK = 128
_D_RBF = 64
_ROWS = 16


def _ln(x, g, b, eps=1e-5):
    m = jnp.mean(x, -1, keepdims=True)
    v = jnp.var(x, -1, keepdims=True)
    return (x - m) * jax.lax.rsqrt(v + eps) * g + b


def _prep_body(msa0_ref, state_ref,
               nmg_ref, nmb_ref, nsg_ref, nsb_ref,
               wn_msa_ref, wn_state_ref, bn_ref,
               ffn_ng_ref, ffn_nb_ref, ffn_w1_ref, ffn_b1_ref,
               ffn_w2_ref, ffn_b2_ref, nng_ref, nnb_ref,
               wm_src_ref, wm_dst_ref, bm_ref,
               nw1_ref, nw2_ref):
    seq = _ln(msa0_ref[...], nmg_ref[...], nmb_ref[...])
    stn = _ln(state_ref[...], nsg_ref[...], nsb_ref[...])
    node = (jnp.dot(seq, wn_msa_ref[...], preferred_element_type=jnp.float32)
            + jnp.dot(stn, wn_state_ref[...], preferred_element_type=jnp.float32)
            + bn_ref[...])
    h = _ln(node, ffn_ng_ref[...], ffn_nb_ref[...])
    h = jax.nn.relu(jnp.dot(h, ffn_w1_ref[...], preferred_element_type=jnp.float32)
                    + ffn_b1_ref[...])
    node = node + jnp.dot(h, ffn_w2_ref[...], preferred_element_type=jnp.float32) + ffn_b2_ref[...]
    node = _ln(node, nng_ref[...], nnb_ref[...])
    nw1_ref[...] = jnp.dot(node, wm_src_ref[...], preferred_element_type=jnp.float32)
    nw2_ref[...] = jnp.dot(node, wm_dst_ref[...], preferred_element_type=jnp.float32) + bm_ref[...]


def _main_body(pair_ref, nbr_ref, msa0_ref, p_ref, tbl_ref, nw2_ref,
               epc_ref, m3_ref, mj_ref, msum_ref,
               pg_ref, pb_ref,
               wp_ref, wr_ref, wnb_ref, be_ref,
               fe_ng_ref, fe_nb_ref, fe_w1_ref, fe_b1_ref, fe_w2_ref, fe_b2_ref,
               neg_ref, neb_ref,
               wm_edge_ref,
               wl0_ref, bl0_ref, wce0_ref, bce0_ref, wce1_ref, bce1_ref,
               s0g_ref, s0b_ref, sig_ref, sib_ref,
               ws0_ref, bs0_ref, wsi_ref, bsi_ref,
               w1_ref, b1_ref, w2_ref, b2_ref, w3_ref, b3_ref, w4_ref, b4_ref,
               wo_ref, bo_ref,
               state_out_ref, xyz_out_ref, alpha_out_ref, quat_out_ref):
    pid = pl.program_id(0)
    R = _ROWS
    EB = R * _TOPK

    nbrb = nbr_ref[...]
    jidx = jax.lax.broadcasted_iota(jnp.int32, (R, _TOPK, _L), 2)
    oh = (nbrb[:, :, None] == jidx).astype(jnp.float32)

    pairb = pair_ref[0]
    tbl = tbl_ref[...]
    gp_parts = []
    gt_parts = []
    for i in range(R):
        ohi = oh[i]
        gp_parts.append(jnp.dot(ohi, pairb[i], preferred_element_type=jnp.float32))
        gt_parts.append(jnp.dot(ohi, tbl, preferred_element_type=jnp.float32))
    gp = jnp.concatenate(gp_parts, axis=0)
    gt = jnp.concatenate(gt_parts, axis=0)
    gsml = gt[:, 0:13]
    gnw1 = gt[:, 128:192]

    pblk = p_ref[pl.ds(pid * R, R), :]
    ca_i = pblk[:, 9:12]
    ca_e = jnp.broadcast_to(ca_i[:, None, :], (R, _TOPK, 3)).reshape(EB, 3)

    caE = jnp.dot(ca_e, epc_ref[...], preferred_element_type=jnp.float32)
    gb13 = gsml - caE
    gb12 = gb13[:, 0:12]
    d2 = jnp.dot(gb13 * gb13, m3_ref[...], preferred_element_type=jnp.float32)
    d = jnp.sqrt(d2 + 1e-8)
    gsrc = jnp.dot(gsml, mj_ref[...], preferred_element_type=jnp.float32)

    mu = (jax.lax.broadcasted_iota(jnp.int32, (1, _D_RBF), 1).astype(jnp.float32)
          * np.float32(20.0 / (_D_RBF - 1)) + 2.0)
    sigma = (22.0 - 2.0) / _D_RBF
    z = (d - mu) / sigma
    rbf = jnp.exp(-z * z)

    re = jax.lax.broadcasted_iota(jnp.int32, (EB, 1), 0)
    ii = ((re >> 7) + R * pid).astype(jnp.float32)
    sep = gsrc - ii
    nbh = jnp.sign(sep) * jnp.log(jnp.abs(sep) + 1.0)

    pairn = _ln(gp, pg_ref[...], pb_ref[...])
    e1 = (jnp.dot(pairn, wp_ref[...], preferred_element_type=jnp.float32)
          + jnp.dot(rbf, wr_ref[...], preferred_element_type=jnp.float32)
          + nbh * wnb_ref[...]
          + be_ref[...])
    he = _ln(e1, fe_ng_ref[...], fe_nb_ref[...])
    he = jax.nn.relu(jnp.dot(he, fe_w1_ref[...], preferred_element_type=jnp.float32)
                     + fe_b1_ref[...])
    e2 = e1 + jnp.dot(he, fe_w2_ref[...], preferred_element_type=jnp.float32) + fe_b2_ref[...]
    edge = _ln(e2, neg_ref[...], neb_ref[...])

    nw2b = nw2_ref[pl.ds(pid * R, R), :]
    nw2e = jnp.broadcast_to(nw2b[:, None, :], (R, _TOPK, 64)).reshape(EB, 64)
    h = jax.nn.relu(jnp.dot(edge, wm_edge_ref[...], preferred_element_type=jnp.float32)
                    + gnw1 + nw2e)

    coefE0 = (jnp.dot(h, wce0_ref[...], preferred_element_type=jnp.float32)
              + bce0_ref[...])
    coefE1 = (jnp.dot(h, wce1_ref[...], preferred_element_type=jnp.float32)
              + bce1_ref[...])
    lm0 = jnp.dot(coefE0 * gb12, msum_ref[...], preferred_element_type=jnp.float32)
    lm1 = jnp.dot(coefE1 * gb12, msum_ref[...], preferred_element_type=jnp.float32)

    ri = jax.lax.broadcasted_iota(jnp.int32, (R, EB), 0)
    ci = jax.lax.broadcasted_iota(jnp.int32, (R, EB), 1)
    sel = ((ci >> 7) == ri).astype(jnp.float32)
    hsum = jnp.dot(sel, h, preferred_element_type=jnp.float32)
    lsum0 = jnp.dot(sel, lm0, preferred_element_type=jnp.float32)
    lsum1 = jnp.dot(sel, lm1, preferred_element_type=jnp.float32)

    l0 = (jnp.dot(hsum, wl0_ref[...], preferred_element_type=jnp.float32)
          * (1.0 / _TOPK) + bl0_ref[...])
    state_out_ref[...] = l0

    T = lsum0 * (1.0 / (_TOPK * 10.0))
    Rv = lsum1 * (1.0 / (_TOPK * 100.0))

    qn = jnp.sqrt(1.0 + jnp.sum(Rv * Rv, axis=1, keepdims=True))
    qA = 1.0 / qn
    qB = Rv[:, 0:1] / qn
    qC = Rv[:, 1:2] / qn
    qD = Rv[:, 2:3] / qn
    r = [[qA * qA + qB * qB - qC * qC - qD * qD, 2 * qB * qC - 2 * qA * qD, 2 * qB * qD + 2 * qA * qC],
         [2 * qB * qC + 2 * qA * qD, qA * qA - qB * qB + qC * qC - qD * qD, 2 * qC * qD - 2 * qA * qB],
         [2 * qB * qD - 2 * qA * qC, 2 * qC * qD + 2 * qA * qB, qA * qA - qB * qB - qC * qC + qD * qD]]
    v = pblk[:, 0:9]
    cols = []
    for a in range(3):
        for c in range(3):
            acc = (r[c][0] * v[:, a * 3 + 0:a * 3 + 1]
                   + r[c][1] * v[:, a * 3 + 1:a * 3 + 2]
                   + r[c][2] * v[:, a * 3 + 2:a * 3 + 3])
            cols.append(acc + ca_i[:, c:c + 1] + T[:, c:c + 1])
    xyz_out_ref[...] = jnp.concatenate(cols, axis=1)
    quat_out_ref[...] = jnp.concatenate([qA, qB, qC, qD], axis=1)

    s = _ln(msa0_ref[...], s0g_ref[...], s0b_ref[...])
    st = _ln(l0, sig_ref[...], sib_ref[...])
    si = (jnp.dot(s, ws0_ref[...], preferred_element_type=jnp.float32) + bs0_ref[...]
          + jnp.dot(st, wsi_ref[...], preferred_element_type=jnp.float32) + bsi_ref[...])
    t = jax.nn.relu(si)
    t = jax.nn.relu(jnp.dot(t, w1_ref[...], preferred_element_type=jnp.float32) + b1_ref[...])
    si = si + jnp.dot(t, w2_ref[...], preferred_element_type=jnp.float32) + b2_ref[...]
    t = jax.nn.relu(si)
    t = jax.nn.relu(jnp.dot(t, w3_ref[...], preferred_element_type=jnp.float32) + b3_ref[...])
    si = si + jnp.dot(t, w4_ref[...], preferred_element_type=jnp.float32) + b4_ref[...]
    alpha_out_ref[...] = (jnp.dot(jax.nn.relu(si), wo_ref[...],
                                  preferred_element_type=jnp.float32) + bo_ref[...])


def _full(shape):
    return pl.BlockSpec(shape, lambda i: tuple(0 for _ in shape))


def kernel(msa, pair, xyz, state, idx, rotation_mask, bond_feats, dist_matrix,
           atom_frames, is_motif, params):
    p = params
    msa0 = msa[0, 0]
    state0 = state[0]
    xyzf = xyz[0]
    cas = xyzf[:, 1, :]
    l1 = (xyzf - cas[:, None, :]).reshape(_L, 9)
    P = jnp.concatenate([l1, cas], axis=1)

    wn = p["embed_node"]["w"]
    wm = p["se3_msg"]["w"]
    we = p["embed_edge"]["w"]

    prep_in = (
        msa0, state0,
        p["norm_msa"]["g"], p["norm_msa"]["b"],
        p["norm_state"]["g"], p["norm_state"]["b"],
        wn[:256], wn[256:], p["embed_node"]["b"],
        p["ff_node"]["ng"], p["ff_node"]["nb"],
        p["ff_node"]["l1"]["w"], p["ff_node"]["l1"]["b"],
        p["ff_node"]["l2"]["w"], p["ff_node"]["l2"]["b"],
        p["norm_node"]["g"], p["norm_node"]["b"],
        wm[0:32], wm[32:64], p["se3_msg"]["b"],
    )
    nw1, nw2 = pl.pallas_call(
        _prep_body,
        out_shape=(
            jax.ShapeDtypeStruct((_L, 64), jnp.float32),
            jax.ShapeDtypeStruct((_L, 64), jnp.float32),
        ),
    )(*prep_in)

    cas_b = xyz[:, :, 1]
    d2_sel = jnp.sum(jnp.square(cas_b[:, :, None, :] - cas_b[:, None, :, :]), -1)
    dg_sel = jnp.sqrt(d2_sel + 1e-8)[0] + jnp.eye(_L) * 1e6
    _, nbr = jax.lax.top_k(-dg_sel, _TOPK)
    nbr = nbr.astype(jnp.int32)

    R = _ROWS
    grid = (_L // R,)
    main_in_specs = [
        pl.BlockSpec((1, R, _L, 128), lambda i: (0, i, 0, 0)),
        pl.BlockSpec((R, _TOPK), lambda i: (i, 0)),
        pl.BlockSpec((R, 256), lambda i: (i, 0)),
        _full((_L, 12)),
        _full((_L, 256)),
        _full((_L, 64)),
        _full((3, 13)), _full((13, 1)), _full((13, 1)),
        _full((12, 3)),
        _full((128,)), _full((128,)),
        _full((128, 32)), _full((64, 32)), _full((1, 32)), _full((32,)),
        _full((32,)), _full((32,)), _full((32, 64)), _full((64,)),
        _full((64, 32)), _full((32,)),
        _full((32,)), _full((32,)),
        _full((32, 64)),
        _full((64, 32)), _full((32,)),
        _full((64, 12)), _full((12,)), _full((64, 12)), _full((12,)),
        _full((256,)), _full((256,)), _full((32,)), _full((32,)),
        _full((256, 128)), _full((128,)), _full((32, 128)), _full((128,)),
        _full((128, 128)), _full((128,)), _full((128, 128)), _full((128,)),
        _full((128, 128)), _full((128,)), _full((128, 128)), _full((128,)),
        _full((128, 20)), _full((20,)),
    ]
    main_out_specs = (
        pl.BlockSpec((R, 32), lambda i: (i, 0)),
        pl.BlockSpec((R, 9), lambda i: (i, 0)),
        pl.BlockSpec((R, 20), lambda i: (i, 0)),
        pl.BlockSpec((R, 4), lambda i: (i, 0)),
    )
    jcolf = jnp.arange(_L, dtype=jnp.float32)[:, None]
    small = jnp.concatenate([l1, cas, jcolf], axis=1)
    small = jnp.pad(small, ((0, 0), (0, 115)))
    tbl = jnp.concatenate([small, jnp.pad(nw1, ((0, 0), (0, 64)))],
                          axis=1)

    eye3 = jnp.eye(3, dtype=jnp.float32)
    epc = jnp.concatenate(
        [jnp.zeros((3, 9), jnp.float32), eye3, jnp.zeros((3, 1), jnp.float32)],
        axis=1)
    m3 = jnp.zeros((13, 1), jnp.float32).at[9:12, 0].set(1.0)
    mj = jnp.zeros((13, 1), jnp.float32).at[12, 0].set(1.0)
    msum = jnp.tile(eye3, (4, 1))
    wc01 = p["se3_coef"]["w"] * 0.1
    bc01 = p["se3_coef"]["b"] * 0.1
    dup0 = np.array([0 * 4 + a for a in range(4) for _ in range(3)])
    dup1 = np.array([1 * 4 + a for a in range(4) for _ in range(3)])
    wce0, wce1 = wc01[:, dup0], wc01[:, dup1]
    bce0, bce1 = bc01[dup0], bc01[dup1]

    main_in = (
        pair, nbr, msa0, P, tbl, nw2,
        epc, m3, mj, msum,
        p["norm_pair"]["g"], p["norm_pair"]["b"],
        we[:128], we[128:192], we[192:193], p["embed_edge"]["b"],
        p["ff_edge"]["ng"], p["ff_edge"]["nb"],
        p["ff_edge"]["l1"]["w"], p["ff_edge"]["l1"]["b"],
        p["ff_edge"]["l2"]["w"], p["ff_edge"]["l2"]["b"],
        p["norm_edge"]["g"], p["norm_edge"]["b"],
        wm[64:96],
        p["se3_l0"]["w"], p["se3_l0"]["b"], wce0, bce0, wce1, bce1,
        p["sc_norm_s0"]["g"], p["sc_norm_s0"]["b"],
        p["sc_norm_si"]["g"], p["sc_norm_si"]["b"],
        p["sc_s0"]["w"], p["sc_s0"]["b"], p["sc_si"]["w"], p["sc_si"]["b"],
        p["sc_1"]["w"], p["sc_1"]["b"], p["sc_2"]["w"], p["sc_2"]["b"],
        p["sc_3"]["w"], p["sc_3"]["b"], p["sc_4"]["w"], p["sc_4"]["b"],
        p["sc_out"]["w"], p["sc_out"]["b"],
    )
    state_new, xyz9, alpha, quat = pl.pallas_call(
        _main_body,
        grid=grid,
        in_specs=main_in_specs,
        out_specs=main_out_specs,
        out_shape=(
            jax.ShapeDtypeStruct((_L, 32), jnp.float32),
            jax.ShapeDtypeStruct((_L, 9), jnp.float32),
            jax.ShapeDtypeStruct((_L, 20), jnp.float32),
            jax.ShapeDtypeStruct((_L, 4), jnp.float32),
        ),
        compiler_params=pltpu.CompilerParams(
            dimension_semantics=("arbitrary",),
        ),
    )(*main_in)

    xyz_new = xyz9.reshape(1, _L, 3, 3)
    state_out = state_new.reshape(1, _L, 32)
    alpha_out = alpha.reshape(1, _L, 10, 2)
    quat_out = quat.reshape(1, _L, 4)
    return xyz_new, state_out, alpha_out, quat_out


# speedup vs baseline: 8.3936x; 1.1173x over previous
import functools

import numpy as np
import jax
import jax.numpy as jnp
from jax.experimental import pallas as pl
from jax.experimental.pallas import tpu as pltpu

_L = 512
_TOPK = 128
_D_RBF = 64
_ROWS = 32


def _ln(x, g, b, eps=1e-5):
    m = jnp.mean(x, -1, keepdims=True)
    v = jnp.var(x, -1, keepdims=True)
    return (x - m) * jax.lax.rsqrt(v + eps) * g + b


def _ln_mm(x, g, b, eps=1e-5):
    k = x.shape[-1]
    j = jnp.full((k, k), 1.0 / k, jnp.float32)
    m = jnp.dot(x, j, preferred_element_type=jnp.float32)
    e2 = jnp.dot(x * x, j, preferred_element_type=jnp.float32)
    v = e2 - m * m
    return (x - m) * jax.lax.rsqrt(v + eps) * g + b


def _prep_body(msa0_ref, state_ref,
               nmg_ref, nmb_ref, nsg_ref, nsb_ref,
               wn_msa_ref, wn_state_ref, bn_ref,
               ffn_ng_ref, ffn_nb_ref, ffn_w1_ref, ffn_b1_ref,
               ffn_w2_ref, ffn_b2_ref, nng_ref, nnb_ref,
               wm_src_ref, wm_dst_ref, bm_ref,
               nw1_ref, nw2_ref):
    seq = _ln(msa0_ref[...], nmg_ref[...], nmb_ref[...])
    stn = _ln(state_ref[...], nsg_ref[...], nsb_ref[...])
    node = (jnp.dot(seq, wn_msa_ref[...], preferred_element_type=jnp.float32)
            + jnp.dot(stn, wn_state_ref[...], preferred_element_type=jnp.float32)
            + bn_ref[...])
    h = _ln(node, ffn_ng_ref[...], ffn_nb_ref[...])
    h = jax.nn.relu(jnp.dot(h, ffn_w1_ref[...], preferred_element_type=jnp.float32)
                    + ffn_b1_ref[...])
    node = node + jnp.dot(h, ffn_w2_ref[...], preferred_element_type=jnp.float32) + ffn_b2_ref[...]
    node = _ln(node, nng_ref[...], nnb_ref[...])
    nw1_ref[...] = jnp.dot(node, wm_src_ref[...], preferred_element_type=jnp.float32)
    nw2_ref[...] = jnp.dot(node, wm_dst_ref[...], preferred_element_type=jnp.float32) + bm_ref[...]


def _main_body(pair_ref, nbr_ref, msa0_ref, p_ref, tbl_ref, nw2_ref,
               epc_ref, m3_ref, mj_ref, msum_ref,
               pg_ref, pb_ref,
               wp_ref, wr_ref, wnb_ref, be_ref,
               fe_ng_ref, fe_nb_ref, fe_w1_ref, fe_b1_ref, fe_w2_ref, fe_b2_ref,
               neg_ref, neb_ref,
               wm_edge_ref,
               wl0_ref, bl0_ref, wce0_ref, bce0_ref, wce1_ref, bce1_ref,
               s0g_ref, s0b_ref, sig_ref, sib_ref,
               ws0_ref, bs0_ref, wsi_ref, bsi_ref,
               w1_ref, b1_ref, w2_ref, b2_ref, w3_ref, b3_ref, w4_ref, b4_ref,
               wo_ref, bo_ref,
               state_out_ref, xyz_out_ref, alpha_out_ref, quat_out_ref):
    pid = pl.program_id(0)
    R = _ROWS
    EB = R * _TOPK

    nbrb = nbr_ref[...]
    jidx = jax.lax.broadcasted_iota(jnp.int32, (R, _TOPK, _L), 2)
    oh = (nbrb[:, :, None] == jidx).astype(jnp.float32)

    pairb = pair_ref[0]
    tbl = tbl_ref[...]
    gp_parts = []
    gt_parts = []
    for i in range(R):
        ohi = oh[i]
        gp_parts.append(jnp.dot(ohi, pairb[i], preferred_element_type=jnp.float32))
        gt_parts.append(jnp.dot(ohi, tbl, preferred_element_type=jnp.float32))
    gp = jnp.concatenate(gp_parts, axis=0)
    gt = jnp.concatenate(gt_parts, axis=0)
    gsml = gt[:, 0:13]
    gnw1 = gt[:, 128:192]

    pblk = p_ref[pl.ds(pid * R, R), :]
    ca_i = pblk[:, 9:12]
    ca_e = jnp.broadcast_to(ca_i[:, None, :], (R, _TOPK, 3)).reshape(EB, 3)

    caE = jnp.dot(ca_e, epc_ref[...], preferred_element_type=jnp.float32)
    gb13 = gsml - caE
    gb12 = gb13[:, 0:12]
    d2 = jnp.dot(gb13 * gb13, m3_ref[...], preferred_element_type=jnp.float32)
    d = jnp.sqrt(d2 + 1e-8)
    gsrc = jnp.dot(gsml, mj_ref[...], preferred_element_type=jnp.float32)

    mu = (jax.lax.broadcasted_iota(jnp.int32, (1, _D_RBF), 1).astype(jnp.float32)
          * np.float32(20.0 / (_D_RBF - 1)) + 2.0)
    sigma = (22.0 - 2.0) / _D_RBF
    z = (d - mu) / sigma
    rbf = jnp.exp(-z * z)

    re = jax.lax.broadcasted_iota(jnp.int32, (EB, 1), 0)
    ii = ((re >> 7) + R * pid).astype(jnp.float32)
    sep = gsrc - ii
    nbh = jnp.sign(sep) * jnp.log(jnp.abs(sep) + 1.0)

    pairn = _ln_mm(gp, pg_ref[...], pb_ref[...])
    e1 = (jnp.dot(pairn, wp_ref[...], preferred_element_type=jnp.float32)
          + jnp.dot(rbf, wr_ref[...], preferred_element_type=jnp.float32)
          + nbh * wnb_ref[...]
          + be_ref[...])
    he = _ln_mm(e1, fe_ng_ref[...], fe_nb_ref[...])
    he = jax.nn.relu(jnp.dot(he, fe_w1_ref[...], preferred_element_type=jnp.float32)
                     + fe_b1_ref[...])
    e2 = e1 + jnp.dot(he, fe_w2_ref[...], preferred_element_type=jnp.float32) + fe_b2_ref[...]
    edge = _ln_mm(e2, neg_ref[...], neb_ref[...])

    nw2b = nw2_ref[pl.ds(pid * R, R), :]
    nw2e = jnp.broadcast_to(nw2b[:, None, :], (R, _TOPK, 64)).reshape(EB, 64)
    h = jax.nn.relu(jnp.dot(edge, wm_edge_ref[...], preferred_element_type=jnp.float32)
                    + gnw1 + nw2e)

    coefE0 = (jnp.dot(h, wce0_ref[...], preferred_element_type=jnp.float32)
              + bce0_ref[...])
    coefE1 = (jnp.dot(h, wce1_ref[...], preferred_element_type=jnp.float32)
              + bce1_ref[...])
    lm0 = jnp.dot(coefE0 * gb12, msum_ref[...], preferred_element_type=jnp.float32)
    lm1 = jnp.dot(coefE1 * gb12, msum_ref[...], preferred_element_type=jnp.float32)

    ri = jax.lax.broadcasted_iota(jnp.int32, (R, EB), 0)
    ci = jax.lax.broadcasted_iota(jnp.int32, (R, EB), 1)
    sel = ((ci >> 7) == ri).astype(jnp.float32)
    hsum = jnp.dot(sel, h, preferred_element_type=jnp.float32)
    lsum0 = jnp.dot(sel, lm0, preferred_element_type=jnp.float32)
    lsum1 = jnp.dot(sel, lm1, preferred_element_type=jnp.float32)

    l0 = (jnp.dot(hsum, wl0_ref[...], preferred_element_type=jnp.float32)
          * (1.0 / _TOPK) + bl0_ref[...])
    state_out_ref[...] = l0

    T = lsum0 * (1.0 / (_TOPK * 10.0))
    Rv = lsum1 * (1.0 / (_TOPK * 100.0))

    qn = jnp.sqrt(1.0 + jnp.sum(Rv * Rv, axis=1, keepdims=True))
    qA = 1.0 / qn
    qB = Rv[:, 0:1] / qn
    qC = Rv[:, 1:2] / qn
    qD = Rv[:, 2:3] / qn
    r = [[qA * qA + qB * qB - qC * qC - qD * qD, 2 * qB * qC - 2 * qA * qD, 2 * qB * qD + 2 * qA * qC],
         [2 * qB * qC + 2 * qA * qD, qA * qA - qB * qB + qC * qC - qD * qD, 2 * qC * qD - 2 * qA * qB],
         [2 * qB * qD - 2 * qA * qC, 2 * qC * qD + 2 * qA * qB, qA * qA - qB * qB - qC * qC + qD * qD]]
    v = pblk[:, 0:9]
    cols = []
    for a in range(3):
        for c in range(3):
            acc = (r[c][0] * v[:, a * 3 + 0:a * 3 + 1]
                   + r[c][1] * v[:, a * 3 + 1:a * 3 + 2]
                   + r[c][2] * v[:, a * 3 + 2:a * 3 + 3])
            cols.append(acc + ca_i[:, c:c + 1] + T[:, c:c + 1])
    xyz_out_ref[...] = jnp.concatenate(cols, axis=1)
    quat_out_ref[...] = jnp.concatenate([qA, qB, qC, qD], axis=1)

    s = _ln(msa0_ref[...], s0g_ref[...], s0b_ref[...])
    st = _ln(l0, sig_ref[...], sib_ref[...])
    si = (jnp.dot(s, ws0_ref[...], preferred_element_type=jnp.float32) + bs0_ref[...]
          + jnp.dot(st, wsi_ref[...], preferred_element_type=jnp.float32) + bsi_ref[...])
    t = jax.nn.relu(si)
    t = jax.nn.relu(jnp.dot(t, w1_ref[...], preferred_element_type=jnp.float32) + b1_ref[...])
    si = si + jnp.dot(t, w2_ref[...], preferred_element_type=jnp.float32) + b2_ref[...]
    t = jax.nn.relu(si)
    t = jax.nn.relu(jnp.dot(t, w3_ref[...], preferred_element_type=jnp.float32) + b3_ref[...])
    si = si + jnp.dot(t, w4_ref[...], preferred_element_type=jnp.float32) + b4_ref[...]
    alpha_out_ref[...] = (jnp.dot(jax.nn.relu(si), wo_ref[...],
                                  preferred_element_type=jnp.float32) + bo_ref[...])


def _full(shape):
    return pl.BlockSpec(shape, lambda i: tuple(0 for _ in shape))


def kernel(msa, pair, xyz, state, idx, rotation_mask, bond_feats, dist_matrix,
           atom_frames, is_motif, params):
    p = params
    msa0 = msa[0, 0]
    state0 = state[0]
    xyzf = xyz[0]
    cas = xyzf[:, 1, :]
    l1 = (xyzf - cas[:, None, :]).reshape(_L, 9)
    P = jnp.concatenate([l1, cas], axis=1)

    wn = p["embed_node"]["w"]
    wm = p["se3_msg"]["w"]
    we = p["embed_edge"]["w"]

    prep_in = (
        msa0, state0,
        p["norm_msa"]["g"], p["norm_msa"]["b"],
        p["norm_state"]["g"], p["norm_state"]["b"],
        wn[:256], wn[256:], p["embed_node"]["b"],
        p["ff_node"]["ng"], p["ff_node"]["nb"],
        p["ff_node"]["l1"]["w"], p["ff_node"]["l1"]["b"],
        p["ff_node"]["l2"]["w"], p["ff_node"]["l2"]["b"],
        p["norm_node"]["g"], p["norm_node"]["b"],
        wm[0:32], wm[32:64], p["se3_msg"]["b"],
    )
    nw1, nw2 = pl.pallas_call(
        _prep_body,
        out_shape=(
            jax.ShapeDtypeStruct((_L, 64), jnp.float32),
            jax.ShapeDtypeStruct((_L, 64), jnp.float32),
        ),
    )(*prep_in)

    cas_b = xyz[:, :, 1]
    d2_sel = jnp.sum(jnp.square(cas_b[:, :, None, :] - cas_b[:, None, :, :]), -1)
    dg_sel = jnp.sqrt(d2_sel + 1e-8)[0] + jnp.eye(_L) * 1e6
    _, nbr = jax.lax.top_k(-dg_sel, _TOPK)
    nbr = nbr.astype(jnp.int32)

    R = _ROWS
    grid = (_L // R,)
    main_in_specs = [
        pl.BlockSpec((1, R, _L, 128), lambda i: (0, i, 0, 0)),
        pl.BlockSpec((R, _TOPK), lambda i: (i, 0)),
        pl.BlockSpec((R, 256), lambda i: (i, 0)),
        _full((_L, 12)),
        _full((_L, 256)),
        _full((_L, 64)),
        _full((3, 13)), _full((13, 1)), _full((13, 1)),
        _full((12, 3)),
        _full((128,)), _full((128,)),
        _full((128, 32)), _full((64, 32)), _full((1, 32)), _full((32,)),
        _full((32,)), _full((32,)), _full((32, 64)), _full((64,)),
        _full((64, 32)), _full((32,)),
        _full((32,)), _full((32,)),
        _full((32, 64)),
        _full((64, 32)), _full((32,)),
        _full((64, 12)), _full((12,)), _full((64, 12)), _full((12,)),
        _full((256,)), _full((256,)), _full((32,)), _full((32,)),
        _full((256, 128)), _full((128,)), _full((32, 128)), _full((128,)),
        _full((128, 128)), _full((128,)), _full((128, 128)), _full((128,)),
        _full((128, 128)), _full((128,)), _full((128, 128)), _full((128,)),
        _full((128, 20)), _full((20,)),
    ]
    main_out_specs = (
        pl.BlockSpec((R, 32), lambda i: (i, 0)),
        pl.BlockSpec((R, 9), lambda i: (i, 0)),
        pl.BlockSpec((R, 20), lambda i: (i, 0)),
        pl.BlockSpec((R, 4), lambda i: (i, 0)),
    )
    jcolf = jnp.arange(_L, dtype=jnp.float32)[:, None]
    small = jnp.concatenate([l1, cas, jcolf], axis=1)
    small = jnp.pad(small, ((0, 0), (0, 115)))
    tbl = jnp.concatenate([small, jnp.pad(nw1, ((0, 0), (0, 64)))],
                          axis=1)

    eye3 = jnp.eye(3, dtype=jnp.float32)
    epc = jnp.concatenate(
        [jnp.zeros((3, 9), jnp.float32), eye3, jnp.zeros((3, 1), jnp.float32)],
        axis=1)
    m3 = jnp.zeros((13, 1), jnp.float32).at[9:12, 0].set(1.0)
    mj = jnp.zeros((13, 1), jnp.float32).at[12, 0].set(1.0)
    msum = jnp.tile(eye3, (4, 1))
    wc01 = p["se3_coef"]["w"] * 0.1
    bc01 = p["se3_coef"]["b"] * 0.1
    dup0 = np.array([0 * 4 + a for a in range(4) for _ in range(3)])
    dup1 = np.array([1 * 4 + a for a in range(4) for _ in range(3)])
    wce0, wce1 = wc01[:, dup0], wc01[:, dup1]
    bce0, bce1 = bc01[dup0], bc01[dup1]

    main_in = (
        pair, nbr, msa0, P, tbl, nw2,
        epc, m3, mj, msum,
        p["norm_pair"]["g"], p["norm_pair"]["b"],
        we[:128], we[128:192], we[192:193], p["embed_edge"]["b"],
        p["ff_edge"]["ng"], p["ff_edge"]["nb"],
        p["ff_edge"]["l1"]["w"], p["ff_edge"]["l1"]["b"],
        p["ff_edge"]["l2"]["w"], p["ff_edge"]["l2"]["b"],
        p["norm_edge"]["g"], p["norm_edge"]["b"],
        wm[64:96],
        p["se3_l0"]["w"], p["se3_l0"]["b"], wce0, bce0, wce1, bce1,
        p["sc_norm_s0"]["g"], p["sc_norm_s0"]["b"],
        p["sc_norm_si"]["g"], p["sc_norm_si"]["b"],
        p["sc_s0"]["w"], p["sc_s0"]["b"], p["sc_si"]["w"], p["sc_si"]["b"],
        p["sc_1"]["w"], p["sc_1"]["b"], p["sc_2"]["w"], p["sc_2"]["b"],
        p["sc_3"]["w"], p["sc_3"]["b"], p["sc_4"]["w"], p["sc_4"]["b"],
        p["sc_out"]["w"], p["sc_out"]["b"],
    )
    state_new, xyz9, alpha, quat = pl.pallas_call(
        _main_body,
        grid=grid,
        in_specs=main_in_specs,
        out_specs=main_out_specs,
        out_shape=(
            jax.ShapeDtypeStruct((_L, 32), jnp.float32),
            jax.ShapeDtypeStruct((_L, 9), jnp.float32),
            jax.ShapeDtypeStruct((_L, 20), jnp.float32),
            jax.ShapeDtypeStruct((_L, 4), jnp.float32),
        ),
        compiler_params=pltpu.CompilerParams(
            dimension_semantics=("arbitrary",),
        ),
    )(*main_in)

    xyz_new = xyz9.reshape(1, _L, 3, 3)
    state_out = state_new.reshape(1, _L, 32)
    alpha_out = alpha.reshape(1, _L, 10, 2)
    quat_out = quat.reshape(1, _L, 4)
    return xyz_new, state_out, alpha_out, quat_out
